# baseline (speedup 1.0000x reference)
import functools

import jax
import jax.numpy as jnp
from jax import lax
from jax.experimental import pallas as pl
from jax.experimental.pallas import tpu as pltpu
from jax.experimental.pallas import tpu_sc as plsc

_R, _C = 4096, 256
_NC, _NS, _L = 2, 16, 16
_NW = _NC * _NS
_RW = _R // _NW


@functools.partial(
    pl.kernel,
    out_type=(
        jax.ShapeDtypeStruct((_R, _C), jnp.int32),
        jax.ShapeDtypeStruct((_R, _C, 3), jnp.float32),
    ),
    mesh=plsc.VectorSubcoreMesh(core_axis_name="c", subcore_axis_name="s"),
    scratch_types=[
        pltpu.VMEM((_RW, _C), jnp.int32),
        pltpu.SemaphoreType.DMA,
    ],
)
def _remap(sp_hbm, coord_hbm, out_hbm, coord_out_hbm, buf, csem):
    wid = lax.axis_index("s") * _NC + lax.axis_index("c")
    r0 = wid * _RW

    ccopy = pltpu.make_async_copy(
        coord_hbm.at[pl.ds(r0, _RW)], coord_out_hbm.at[pl.ds(r0, _RW)], csem
    )
    ccopy.start()

    pltpu.sync_copy(sp_hbm.at[pl.ds(r0, _RW), :], buf)

    neg1 = jnp.full((_L,), -1, jnp.int32)

    def step(r, carry):
        for u in range(_C // _L):
            v = buf[r, pl.ds(u * _L, _L)]
            w = v - 1
            ok = w.astype(jnp.uint32) < jnp.uint32(8)
            buf[r, pl.ds(u * _L, _L)] = jnp.where(ok, w, neg1)
        return carry

    lax.fori_loop(0, _RW, step, 0)
    pltpu.sync_copy(buf, out_hbm.at[pl.ds(r0, _RW), :])
    ccopy.wait()


def kernel(species, coordinates):
    out, coords = _remap(species, coordinates)
    return out, coords


# speedup vs baseline: 136.0141x; 136.0141x over previous
import functools

import jax
import jax.numpy as jnp
from jax import lax
from jax.experimental import pallas as pl
from jax.experimental.pallas import tpu as pltpu
from jax.experimental.pallas import tpu_sc as plsc

_R, _C = 4096, 256
_NC, _NS, _L = 2, 16, 16
_NW = _NC * _NS
_RW = _R // _NW


@functools.partial(
    pl.kernel,
    out_type=jax.ShapeDtypeStruct((_R, _C), jnp.int32),
    mesh=plsc.VectorSubcoreMesh(core_axis_name="c", subcore_axis_name="s"),
    scratch_types=[pltpu.VMEM((_RW, _C), jnp.int32)],
)
def _remap(sp_hbm, out_hbm, buf):
    wid = lax.axis_index("s") * _NC + lax.axis_index("c")
    r0 = wid * _RW
    pltpu.sync_copy(sp_hbm.at[pl.ds(r0, _RW), :], buf)

    neg1 = jnp.full((_L,), -1, jnp.int32)

    def step(r, carry):
        for u in range(_C // _L):
            v = buf[r, pl.ds(u * _L, _L)]
            w = v - 1
            ok = w.astype(jnp.uint32) < jnp.uint32(8)
            buf[r, pl.ds(u * _L, _L)] = jnp.where(ok, w, neg1)
        return carry

    lax.fori_loop(0, _RW, step, 0)
    pltpu.sync_copy(buf, out_hbm.at[pl.ds(r0, _RW), :])


def _copy_body(x_ref, o_ref):
    o_ref[...] = x_ref[...]


_TC_BLOCK_ROWS = 512


def _tc_copy(coords):
    flat = coords.reshape(_R, _C * 3)
    out = pl.pallas_call(
        _copy_body,
        grid=(_R // _TC_BLOCK_ROWS,),
        in_specs=[pl.BlockSpec((_TC_BLOCK_ROWS, _C * 3), lambda i: (i, 0))],
        out_specs=pl.BlockSpec((_TC_BLOCK_ROWS, _C * 3), lambda i: (i, 0)),
        out_shape=jax.ShapeDtypeStruct((_R, _C * 3), jnp.float32),
    )(flat)
    return out.reshape(_R, _C, 3)


def kernel(species, coordinates):
    return _remap(species), _tc_copy(coordinates)


# speedup vs baseline: 484.7556x; 3.5640x over previous
import functools

import jax
import jax.numpy as jnp
from jax import lax
from jax.experimental import pallas as pl
from jax.experimental.pallas import tpu as pltpu
from jax.experimental.pallas import tpu_sc as plsc

_R, _C = 4096, 256
_NC, _NS, _L = 2, 16, 16
_NW = _NC * _NS
_RW = _R // _NW


@functools.partial(
    pl.kernel,
    out_type=jax.ShapeDtypeStruct((_R, _C), jnp.int32),
    mesh=plsc.VectorSubcoreMesh(core_axis_name="c", subcore_axis_name="s"),
    scratch_types=[pltpu.VMEM((_RW, _C), jnp.int32)],
)
def _remap(sp_hbm, out_hbm, buf):
    wid = lax.axis_index("s") * _NC + lax.axis_index("c")
    r0 = wid * _RW
    pltpu.sync_copy(sp_hbm.at[pl.ds(r0, _RW), :], buf)

    neg1 = jnp.full((_L,), -1, jnp.int32)

    def step(r, carry):
        for u in range(_C // _L):
            v = buf[r, pl.ds(u * _L, _L)]
            w = v - 1
            ok = w.astype(jnp.uint32) < jnp.uint32(8)
            buf[r, pl.ds(u * _L, _L)] = jnp.where(ok, w, neg1)
        return carry

    lax.fori_loop(0, _RW, step, 0)
    pltpu.sync_copy(buf, out_hbm.at[pl.ds(r0, _RW), :])


def kernel(species, coordinates):
    coords_out = jnp.copy(coordinates)
    species_b, coords_out = lax.optimization_barrier((species, coords_out))
    return _remap(species_b), coords_out


# speedup vs baseline: 497.8930x; 1.0271x over previous
import functools

import jax
import jax.numpy as jnp
from jax import lax
from jax.experimental import pallas as pl
from jax.experimental.pallas import tpu as pltpu
from jax.experimental.pallas import tpu_sc as plsc

_R, _C = 4096, 256
_NC, _NS, _L = 2, 16, 16
_NW = _NC * _NS
_RW = _R // _NW


@functools.partial(
    pl.kernel,
    out_type=(
        jax.ShapeDtypeStruct((_R, _C), jnp.int32),
        jax.ShapeDtypeStruct((3, _R, _C), jnp.float32),
    ),
    mesh=plsc.VectorSubcoreMesh(core_axis_name="c", subcore_axis_name="s"),
    scratch_types=[
        pltpu.VMEM((_RW, _C), jnp.int32),
        pltpu.VMEM((_RW, _C), jnp.float32),
        pltpu.VMEM((_RW, _C), jnp.float32),
        pltpu.SemaphoreType.DMA,
        pltpu.SemaphoreType.DMA,
        pltpu.SemaphoreType.DMA,
        pltpu.SemaphoreType.DMA,
        pltpu.SemaphoreType.DMA,
        pltpu.SemaphoreType.DMA,
    ],
)
def _remap(sp_hbm, coord_hbm, out_hbm, coord_out_hbm,
           sbuf, cb0, cb1, ssem, osem, ci0, ci1, co0, co1):
    cid = lax.axis_index("c")
    sid = lax.axis_index("s")
    wid = sid * _NC + cid
    r0 = wid * _RW
    cbuf = [cb0, cb1]
    cisem = [ci0, ci1]
    cosem = [co0, co1]

    s_in = pltpu.make_async_copy(sp_hbm.at[pl.ds(r0, _RW), :], sbuf, ssem)
    s_in.start()
    c_in = []
    for p in range(2):
        cp = pltpu.make_async_copy(
            coord_hbm.at[p, pl.ds(r0, _RW), :], cbuf[p], cisem[p])
        cp.start()
        c_in.append(cp)

    neg1 = jnp.full((_L,), -1, jnp.int32)

    def step(r, carry):
        for u in range(_C // _L):
            v = sbuf[r, pl.ds(u * _L, _L)]
            w = v - 1
            ok = w.astype(jnp.uint32) < jnp.uint32(8)
            sbuf[r, pl.ds(u * _L, _L)] = jnp.where(ok, w, neg1)
        return carry

    s_in.wait()
    lax.fori_loop(0, _RW, step, 0)
    s_out = pltpu.make_async_copy(sbuf, out_hbm.at[pl.ds(r0, _RW), :], osem)
    s_out.start()

    c_out = []
    for p in range(3):
        slot = p % 2
        c_in[p].wait()
        ocp = pltpu.make_async_copy(
            cbuf[slot], coord_out_hbm.at[p, pl.ds(r0, _RW), :], cosem[slot])
        ocp.start()
        c_out.append(ocp)
        if p + 2 < 3:
            c_out[p].wait()
            cp = pltpu.make_async_copy(
                coord_hbm.at[p + 2, pl.ds(r0, _RW), :], cbuf[slot], cisem[slot])
            cp.start()
            c_in.append(cp)

    c_out[1].wait()
    c_out[2].wait()
    s_out.wait()


def kernel(species, coordinates):
    coords3 = jnp.transpose(coordinates, (2, 0, 1))
    out, coords_out = _remap(species, coords3)
    return out, jnp.transpose(coords_out, (1, 2, 0))


# speedup vs baseline: 512.4093x; 1.0292x over previous
import functools

import jax
import jax.numpy as jnp
from jax import lax
from jax.experimental import pallas as pl
from jax.experimental.pallas import tpu as pltpu
from jax.experimental.pallas import tpu_sc as plsc

_R, _C = 4096, 256
_NC, _NS, _L = 2, 16, 16
_NW = _NC * _NS
_RW = _R // _NW


@functools.partial(
    pl.kernel,
    out_type=(
        jax.ShapeDtypeStruct((_R, _C), jnp.int32),
        jax.ShapeDtypeStruct((3, _R, _C), jnp.float32),
    ),
    mesh=plsc.VectorSubcoreMesh(core_axis_name="c", subcore_axis_name="s"),
    scratch_types=[
        pltpu.VMEM((_RW, _C), jnp.int32),
        pltpu.VMEM((_RW, _C), jnp.float32),
        pltpu.VMEM((_RW, _C), jnp.float32),
        pltpu.SemaphoreType.DMA,
        pltpu.SemaphoreType.DMA,
        pltpu.SemaphoreType.DMA,
        pltpu.SemaphoreType.DMA,
        pltpu.SemaphoreType.DMA,
        pltpu.SemaphoreType.DMA,
    ],
)
def _remap(sp_hbm, coord_hbm, out_hbm, coord_out_hbm,
           sbuf, cb0, cb1, ssem, osem, ci0, ci1, co0, co1):
    cid = lax.axis_index("c")
    sid = lax.axis_index("s")
    wid = sid * _NC + cid
    r0 = wid * _RW
    cbuf = [cb0, cb1]
    cisem = [ci0, ci1]
    cosem = [co0, co1]

    s_in = pltpu.make_async_copy(sp_hbm.at[pl.ds(r0, _RW), :], sbuf, ssem)
    s_in.start()
    c_in = []
    for p in range(2):
        cp = pltpu.make_async_copy(
            coord_hbm.at[p, pl.ds(r0, _RW), :], cbuf[p], cisem[p])
        cp.start()
        c_in.append(cp)

    neg1 = jnp.full((_L,), -1, jnp.int32)

    def step(r, carry):
        for u in range(_C // _L):
            v = sbuf[r, pl.ds(u * _L, _L)]
            w = v - 1
            ok = w.astype(jnp.uint32) < jnp.uint32(8)
            sbuf[r, pl.ds(u * _L, _L)] = jnp.where(ok, w, neg1)
        return carry

    s_in.wait()
    lax.fori_loop(0, _RW, step, 0)
    s_out = pltpu.make_async_copy(sbuf, out_hbm.at[pl.ds(r0, _RW), :], osem)
    s_out.start()

    c_out = []
    for p in range(2):
        c_in[p].wait()
        ocp = pltpu.make_async_copy(
            cbuf[p], coord_out_hbm.at[p, pl.ds(r0, _RW), :], cosem[p])
        ocp.start()
        c_out.append(ocp)
    c_out[0].wait()
    cin2 = pltpu.make_async_copy(
        coord_hbm.at[2, pl.ds(r0, _RW), :], cbuf[0], cisem[0])
    cin2.start()
    cin2.wait()
    cout2 = pltpu.make_async_copy(
        cbuf[0], coord_out_hbm.at[2, pl.ds(r0, _RW), :], cosem[0])
    cout2.start()

    c_out[1].wait()
    cout2.wait()
    s_out.wait()


def kernel(species, coordinates):
    coords3 = jnp.transpose(coordinates, (2, 0, 1))
    out, coords_out = _remap(species, coords3)
    return out, jnp.transpose(coords_out, (1, 2, 0))
